# Initial kernel scaffold; baseline (speedup 1.0000x reference)
import jax
import jax.numpy as jnp
from jax.experimental import pallas as pl


def kernel(dense_points_data, dense_points_idx, sparse_points_data, sparse_points_idx, pcd, W, gamma, beta):
    raise NotImplementedError("write your pallas kernel here")


# speedup vs baseline: 807.3402x; 807.3402x over previous
import functools

import jax
import jax.numpy as jnp
from jax import lax
from jax.experimental import pallas as pl
from jax.experimental.pallas import tpu as pltpu
from jax.experimental.pallas import tpu_sc as plsc

_B, _N, _ND, _NS = 4, 16384, 4096, 1024
_CD, _CS, _VD = 128, 256, 128
_T = 512
_NT = _ND // _T
_TB = 2048

_SC_CORES, _SC_SUBCORES = 2, 16
_NW = _SC_CORES * _SC_SUBCORES
_ROWS = _B * (_ND + _NS)
_CHUNK = 128
_NCHUNK = _ROWS // (_NW * _CHUNK)


def _sc_gather_rows(table, idx3):
    mesh = plsc.VectorSubcoreMesh(core_axis_name="c", subcore_axis_name="s")

    @functools.partial(
        pl.kernel,
        mesh=mesh,
        out_type=jax.ShapeDtypeStruct((_NW, _NCHUNK, _CHUNK, 8), jnp.float32),
        scratch_types=[
            pltpu.VMEM((_NCHUNK, _CHUNK), jnp.int32),
            pltpu.VMEM((_NCHUNK, _CHUNK, 8), jnp.float32),
            pltpu.SemaphoreType.DMA,
        ],
        compiler_params=pltpu.CompilerParams(use_tc_tiling_on_sc=False),
    )
    def k(table_hbm, idx_hbm, out_hbm, idx_v, rows_v, sem):
        wid = lax.axis_index("s") * _SC_CORES + lax.axis_index("c")
        pltpu.sync_copy(idx_hbm.at[wid], idx_v)
        handles = [
            pltpu.async_copy(table_hbm.at[idx_v.at[c]], rows_v.at[c], sem)
            for c in range(_NCHUNK)
        ]
        for h in handles:
            h.wait()
        pltpu.sync_copy(rows_v, out_hbm.at[wid])

    return k(table, idx3)


def _tc_main_body(dxyz_ref, sxyz_ref, dd_ref, sd_ref, w_ref, y_ref, st_ref, p_ref):
    i = pl.program_id(1)
    first = jnp.logical_and(pl.program_id(0) == 0, i == 0)

    @pl.when(i == 0)
    def _():
        p_ref[...] = lax.dot_general(
            w_ref[:, _CD:], sd_ref[0],
            (((1,), (0,)), ((), ())),
            precision=lax.Precision.HIGHEST,
            preferred_element_type=jnp.float32)

    dxyz_t = dxyz_ref[0]
    sxyz_r = sxyz_ref[0]

    dn = jnp.sum(dxyz_t * dxyz_t, axis=0, keepdims=True)
    sn = jnp.sum(sxyz_r * sxyz_r, axis=1, keepdims=True)
    cross = lax.dot_general(
        sxyz_r.astype(jnp.bfloat16), dxyz_t.astype(jnp.bfloat16),
        (((1,), (0,)), ((), ())),
        preferred_element_type=jnp.float32)
    d2 = jnp.maximum((sn + dn) - 2.0 * cross, 0.0)

    iota0 = lax.broadcasted_iota(jnp.int32, (_NS, _T), 0)
    work = d2
    vals, idxs = [], []
    for k in range(3):
        m = jnp.min(work, axis=0, keepdims=True)
        hit = work == m
        ik = jnp.min(jnp.where(hit, iota0, _NS), axis=0, keepdims=True)
        vals.append(m)
        idxs.append(ik)
        if k < 2:
            work = jnp.where(iota0 == ik, jnp.float32(jnp.inf), work)

    w0 = 1.0 / (vals[0] + 1e-08)
    w1 = 1.0 / (vals[1] + 1e-08)
    w2 = 1.0 / (vals[2] + 1e-08)
    ws = w0 + w1 + w2
    w0, w1, w2 = w0 / ws, w1 / ws, w2 / ws

    wsel = jnp.where(iota0 == idxs[0], w0, 0.0)
    wsel = wsel + jnp.where(iota0 == idxs[1], w1, 0.0)
    wsel = wsel + jnp.where(iota0 == idxs[2], w2, 0.0)

    yd = lax.dot_general(
        w_ref[:, :_CD].astype(jnp.bfloat16), dd_ref[0].astype(jnp.bfloat16),
        (((1,), (0,)), ((), ())),
        preferred_element_type=jnp.float32)
    ys = lax.dot_general(
        p_ref[...].astype(jnp.bfloat16), wsel.astype(jnp.bfloat16),
        (((1,), (0,)), ((), ())),
        preferred_element_type=jnp.float32)
    y = yd + ys
    y_ref[0] = y

    s1 = jnp.sum(y, axis=1, keepdims=True)
    s2 = jnp.sum(y * y, axis=1, keepdims=True)
    stacked = jnp.concatenate([s1, s2], axis=1)

    @pl.when(first)
    def _():
        st_ref[...] = stacked

    @pl.when(jnp.logical_not(first))
    def _():
        st_ref[...] = st_ref[...] + stacked


def _run_tc_main(dxyz_t, sxyz_r, dd, sd, w, *, interpret=False):
    return pl.pallas_call(
        _tc_main_body,
        grid=(_B, _NT),
        in_specs=[
            pl.BlockSpec((1, 8, _T), lambda b, i: (b, 0, i)),
            pl.BlockSpec((1, _NS, 8), lambda b, i: (b, 0, 0)),
            pl.BlockSpec((1, _CD, _T), lambda b, i: (b, 0, i)),
            pl.BlockSpec((1, _CS, _NS), lambda b, i: (b, 0, 0)),
            pl.BlockSpec((_VD, 3 * _VD), lambda b, i: (0, 0)),
        ],
        out_specs=[
            pl.BlockSpec((1, _VD, _T), lambda b, i: (b, 0, i)),
            pl.BlockSpec((_VD, 2), lambda b, i: (0, 0)),
        ],
        out_shape=[
            jax.ShapeDtypeStruct((_B, _VD, _ND), jnp.float32),
            jax.ShapeDtypeStruct((_VD, 2), jnp.float32),
        ],
        scratch_shapes=[pltpu.VMEM((_VD, _NS), jnp.float32)],
        interpret=interpret,
    )(dxyz_t, sxyz_r, dd, sd, w)


def _bn_body(y_ref, pars_ref, o_ref):
    sc = pars_ref[:, 0:1]
    sh = pars_ref[:, 1:2]
    t = y_ref[0] * sc + sh
    o_ref[0] = jnp.where(t >= 0, t, 0.2 * t)


def _run_bn(y, pars, *, interpret=False):
    return pl.pallas_call(
        _bn_body,
        grid=(_B, _ND // _TB),
        in_specs=[
            pl.BlockSpec((1, _VD, _TB), lambda b, i: (b, 0, i)),
            pl.BlockSpec((_VD, 2), lambda b, i: (0, 0)),
        ],
        out_specs=pl.BlockSpec((1, _VD, _TB), lambda b, i: (b, 0, i)),
        out_shape=jax.ShapeDtypeStruct((_B, _VD, _ND), jnp.float32),
        interpret=interpret,
    )(y, pars)


def kernel(dense_points_data, dense_points_idx, sparse_points_data,
           sparse_points_idx, pcd, W, gamma, beta):
    f32 = jnp.float32
    pcd_rows = jnp.swapaxes(pcd, 1, 2)
    table = jnp.concatenate(
        [pcd_rows, jnp.zeros((_B, _N, 5), f32)], axis=2).reshape(_B * _N, 8)
    offs = (jnp.arange(_B, dtype=jnp.int32) * _N)[:, None]
    all_idx = jnp.concatenate([
        (dense_points_idx + offs).reshape(-1),
        (sparse_points_idx + offs).reshape(-1),
    ])
    idx3 = all_idx.reshape(_NW, _NCHUNK, _CHUNK)

    rows = _sc_gather_rows(table, idx3).reshape(_ROWS, 8)
    dxyz_t = rows[:_B * _ND].reshape(_B, _ND, 8).transpose(0, 2, 1)
    sxyz_r = rows[_B * _ND:].reshape(_B, _NS, 8)

    y, stats = _run_tc_main(dxyz_t, sxyz_r, dense_points_data,
                            sparse_points_data, W)

    cnt = jnp.float32(_B * _ND)
    mean = stats[:, 0] / cnt
    var = stats[:, 1] / cnt - mean * mean
    scale = gamma / jnp.sqrt(var + 1e-05)
    shift = beta - mean * scale
    pars = jnp.stack([scale, shift], axis=1)

    out = _run_bn(y, pars)
    return (out, dense_points_idx)
